# Initial kernel scaffold; baseline (speedup 1.0000x reference)
import jax
import jax.numpy as jnp
from jax.experimental import pallas as pl


def kernel(query_t, vid_timestamps, time_masks, k):
    raise NotImplementedError("write your pallas kernel here")


# speedup vs baseline: 306.2163x; 306.2163x over previous
import functools

import jax
import jax.numpy as jnp
from jax import lax
from jax.experimental import pallas as pl
from jax.experimental.pallas import tpu as pltpu
from jax.experimental.pallas import tpu_sc as plsc

V, T, N, K = 16, 4096, 1024, 16
LANES = 16
NWORKERS = 32
QPW = V * N // NWORKERS
CHUNKS = QPW // LANES


def _tec_body(q_hbm, a_hbm, len_hbm, out_hbm, q_v, a_v, len_v, out_v):
    c = lax.axis_index("c")
    s = lax.axis_index("s")
    wid = s * 2 + c
    vid = wid // 2
    half = wid % 2
    q0 = half * QPW

    pltpu.sync_copy(a_hbm.at[vid], a_v)
    pltpu.sync_copy(q_hbm.at[vid, pl.ds(q0, QPW)], q_v)
    pltpu.sync_copy(len_hbm, len_v)

    lane = lax.iota(jnp.int32, LANES)
    l_vec = plsc.load_gather(len_v, [jnp.full((LANES,), vid, jnp.int32)])
    inf = jnp.float32(jnp.inf)

    def chunk(i, carry):
        qidx = i * LANES + lane
        q = plsc.load_gather(q_v, [qidx])
        qq = q * q

        pos = jnp.zeros((LANES,), jnp.int32)
        b = T // 2
        while b >= 1:
            av = plsc.load_gather(a_v, [pos + (b - 1)])
            pos = jnp.where(av < q, pos + b, pos)
            b //= 2
        pos = jnp.minimum(pos, l_vec)

        lp = pos - 1
        rp = pos
        for st in range(K):
            la = plsc.load_gather(a_v, [jnp.maximum(lp, 0)])
            ra = plsc.load_gather(a_v, [jnp.minimum(rp, T - 1)])
            dl = jnp.where(lp >= 0, (qq + la * la) - 2.0 * (q * la), inf)
            dr = jnp.where(rp < l_vec, (qq + ra * ra) - 2.0 * (q * ra), inf)
            tl = dl <= dr
            plsc.store_scatter(
                out_v, [qidx, jnp.full((LANES,), st, jnp.int32)],
                jnp.where(tl, lp, rp))
            lp = jnp.where(tl, lp - 1, lp)
            rp = jnp.where(tl, rp, rp + 1)
        return carry

    lax.fori_loop(0, CHUNKS, chunk, 0)
    pltpu.sync_copy(out_v, out_hbm.at[vid, pl.ds(q0, QPW)])


def kernel(query_t, vid_timestamps, time_masks, k):
    q = query_t[..., 0]
    a = vid_timestamps[..., 0]
    lengths = jnp.sum(time_masks[:, :, 0], axis=1, dtype=jnp.int32)

    mesh = plsc.VectorSubcoreMesh(core_axis_name="c", subcore_axis_name="s")
    fn = pl.kernel(
        _tec_body,
        out_type=jax.ShapeDtypeStruct((V, N, K), jnp.int32),
        mesh=mesh,
        compiler_params=pltpu.CompilerParams(needs_layout_passes=False),
        scratch_types=[
            pltpu.VMEM((QPW,), jnp.float32),
            pltpu.VMEM((T,), jnp.float32),
            pltpu.VMEM((V,), jnp.int32),
            pltpu.VMEM((QPW, K), jnp.int32),
        ],
    )
    return fn(q, a, lengths)


# speedup vs baseline: 329.4177x; 1.0758x over previous
import functools

import jax
import jax.numpy as jnp
from jax import lax
from jax.experimental import pallas as pl
from jax.experimental.pallas import tpu as pltpu
from jax.experimental.pallas import tpu_sc as plsc

V, T, N, K = 16, 4096, 1024, 16
LANES = 16
NWORKERS = 32
QPW = V * N // NWORKERS
CHUNKS = QPW // LANES


def _tec_body(q_hbm, a_hbm, len_hbm, out_hbm, q_v, a_v, len_v, out_v):
    c = lax.axis_index("c")
    s = lax.axis_index("s")
    wid = s * 2 + c
    vid = wid // 2
    half = wid % 2
    q0 = half * QPW

    pltpu.sync_copy(a_hbm.at[vid], a_v)
    pltpu.sync_copy(q_hbm.at[vid, pl.ds(q0, QPW)], q_v)
    pltpu.sync_copy(len_hbm, len_v)

    lane = lax.iota(jnp.int32, LANES)
    l_vec = plsc.load_gather(len_v, [jnp.full((LANES,), vid, jnp.int32)])
    inf = jnp.float32(jnp.inf)

    ILP = 2

    def chunk(i, carry):
        qidx = [i * (LANES * ILP) + j * LANES + lane for j in range(ILP)]
        q = [plsc.load_gather(q_v, [qidx[j]]) for j in range(ILP)]
        qq = [q[j] * q[j] for j in range(ILP)]

        pos = [jnp.zeros((LANES,), jnp.int32) for _ in range(ILP)]
        b = T // 2
        while b >= 1:
            for j in range(ILP):
                av = plsc.load_gather(a_v, [pos[j] + (b - 1)])
                pos[j] = jnp.where(av < q[j], pos[j] + b, pos[j])
            b //= 2
        pos = [jnp.minimum(p, l_vec) for p in pos]

        lp = [p - 1 for p in pos]
        rp = list(pos)
        for st in range(K):
            for j in range(ILP):
                la = plsc.load_gather(a_v, [jnp.maximum(lp[j], 0)])
                ra = plsc.load_gather(a_v, [jnp.minimum(rp[j], T - 1)])
                dl = jnp.where(lp[j] >= 0, (qq[j] + la * la) - 2.0 * (q[j] * la), inf)
                dr = jnp.where(rp[j] < l_vec, (qq[j] + ra * ra) - 2.0 * (q[j] * ra), inf)
                tl = dl <= dr
                plsc.store_scatter(
                    out_v, [qidx[j], jnp.full((LANES,), st, jnp.int32)],
                    jnp.where(tl, lp[j], rp[j]))
                lp[j] = jnp.where(tl, lp[j] - 1, lp[j])
                rp[j] = jnp.where(tl, rp[j], rp[j] + 1)
        return carry

    lax.fori_loop(0, CHUNKS // ILP, chunk, 0)
    pltpu.sync_copy(out_v, out_hbm.at[vid, pl.ds(q0, QPW)])


def kernel(query_t, vid_timestamps, time_masks, k):
    q = query_t[..., 0]
    a = vid_timestamps[..., 0]
    lengths = jnp.sum(time_masks[:, :, 0], axis=1, dtype=jnp.int32)

    mesh = plsc.VectorSubcoreMesh(core_axis_name="c", subcore_axis_name="s")
    fn = pl.kernel(
        _tec_body,
        out_type=jax.ShapeDtypeStruct((V, N, K), jnp.int32),
        mesh=mesh,
        compiler_params=pltpu.CompilerParams(needs_layout_passes=False),
        scratch_types=[
            pltpu.VMEM((QPW,), jnp.float32),
            pltpu.VMEM((T,), jnp.float32),
            pltpu.VMEM((V,), jnp.int32),
            pltpu.VMEM((QPW, K), jnp.int32),
        ],
    )
    return fn(q, a, lengths)


# speedup vs baseline: 330.1726x; 1.0023x over previous
import functools

import jax
import jax.numpy as jnp
from jax import lax
from jax.experimental import pallas as pl
from jax.experimental.pallas import tpu as pltpu
from jax.experimental.pallas import tpu_sc as plsc

V, T, N, K = 16, 4096, 1024, 16
LANES = 16
NWORKERS = 32
QPW = V * N // NWORKERS
CHUNKS = QPW // LANES


def _tec_body(q_hbm, a_hbm, len_hbm, out_hbm, q_v, a_v, len_v, out_v):
    c = lax.axis_index("c")
    s = lax.axis_index("s")
    wid = s * 2 + c
    vid = wid // 2
    half = wid % 2
    q0 = half * QPW

    pltpu.sync_copy(a_hbm.at[vid], a_v)
    pltpu.sync_copy(q_hbm.at[vid, pl.ds(q0, QPW)], q_v)
    pltpu.sync_copy(len_hbm, len_v)

    lane = lax.iota(jnp.int32, LANES)
    l_vec = plsc.load_gather(len_v, [jnp.full((LANES,), vid, jnp.int32)])
    inf = jnp.float32(jnp.inf)

    ILP = 4

    def chunk(i, carry):
        qidx = [i * (LANES * ILP) + j * LANES + lane for j in range(ILP)]
        q = [plsc.load_gather(q_v, [qidx[j]]) for j in range(ILP)]
        qq = [q[j] * q[j] for j in range(ILP)]

        pos = [jnp.zeros((LANES,), jnp.int32) for _ in range(ILP)]
        b = T // 2
        while b >= 1:
            for j in range(ILP):
                av = plsc.load_gather(a_v, [pos[j] + (b - 1)])
                pos[j] = jnp.where(av < q[j], pos[j] + b, pos[j])
            b //= 2
        pos = [jnp.minimum(p, l_vec) for p in pos]

        lp = [p - 1 for p in pos]
        rp = list(pos)
        for st in range(K):
            for j in range(ILP):
                la = plsc.load_gather(a_v, [jnp.maximum(lp[j], 0)])
                ra = plsc.load_gather(a_v, [jnp.minimum(rp[j], T - 1)])
                dl = jnp.where(lp[j] >= 0, (qq[j] + la * la) - 2.0 * (q[j] * la), inf)
                dr = jnp.where(rp[j] < l_vec, (qq[j] + ra * ra) - 2.0 * (q[j] * ra), inf)
                tl = dl <= dr
                plsc.store_scatter(
                    out_v, [qidx[j], jnp.full((LANES,), st, jnp.int32)],
                    jnp.where(tl, lp[j], rp[j]))
                lp[j] = jnp.where(tl, lp[j] - 1, lp[j])
                rp[j] = jnp.where(tl, rp[j], rp[j] + 1)
        return carry

    lax.fori_loop(0, CHUNKS // ILP, chunk, 0)
    pltpu.sync_copy(out_v, out_hbm.at[vid, pl.ds(q0, QPW)])


def kernel(query_t, vid_timestamps, time_masks, k):
    q = query_t[..., 0]
    a = vid_timestamps[..., 0]
    lengths = jnp.sum(time_masks[:, :, 0], axis=1, dtype=jnp.int32)

    mesh = plsc.VectorSubcoreMesh(core_axis_name="c", subcore_axis_name="s")
    fn = pl.kernel(
        _tec_body,
        out_type=jax.ShapeDtypeStruct((V, N, K), jnp.int32),
        mesh=mesh,
        compiler_params=pltpu.CompilerParams(needs_layout_passes=False),
        scratch_types=[
            pltpu.VMEM((QPW,), jnp.float32),
            pltpu.VMEM((T,), jnp.float32),
            pltpu.VMEM((V,), jnp.int32),
            pltpu.VMEM((QPW, K), jnp.int32),
        ],
    )
    return fn(q, a, lengths)


# speedup vs baseline: 347.1019x; 1.0513x over previous
import functools

import jax
import jax.numpy as jnp
from jax import lax
from jax.experimental import pallas as pl
from jax.experimental.pallas import tpu as pltpu
from jax.experimental.pallas import tpu_sc as plsc

V, T, N, K = 16, 4096, 1024, 16
LANES = 16
NWORKERS = 32
QPW = V * N // NWORKERS
CHUNKS = QPW // LANES


def _tec_body(q_hbm, a_hbm, len_hbm, out_hbm, q_v, a_v, len_v, out_v):
    c = lax.axis_index("c")
    s = lax.axis_index("s")
    wid = s * 2 + c
    vid = wid // 2
    half = wid % 2
    q0 = half * QPW

    pltpu.sync_copy(a_hbm.at[vid], a_v)
    pltpu.sync_copy(q_hbm.at[vid, pl.ds(q0, QPW)], q_v)
    pltpu.sync_copy(len_hbm, len_v)

    lane = lax.iota(jnp.int32, LANES)
    l_vec = plsc.load_gather(len_v, [jnp.full((LANES,), vid, jnp.int32)])
    inf = jnp.float32(jnp.inf)

    ILP = 2

    def chunk(i, carry):
        qidx = [i * (LANES * ILP) + j * LANES + lane for j in range(ILP)]
        q = [plsc.load_gather(q_v, [qidx[j]]) for j in range(ILP)]
        qq = [q[j] * q[j] for j in range(ILP)]

        pos = [jnp.zeros((LANES,), jnp.int32) for _ in range(ILP)]
        b = T // 2
        while b >= 1:
            for j in range(ILP):
                av = plsc.load_gather(a_v, [pos[j] + (b - 1)])
                pos[j] = jnp.where(av < q[j], pos[j] + b, pos[j])
            b //= 2
        pos = [jnp.minimum(p, l_vec) for p in pos]

        lp = [p - 1 for p in pos]
        rp = list(pos)
        for st in range(K):
            for j in range(ILP):
                la = plsc.load_gather(a_v, [jnp.maximum(lp[j], 0)])
                ra = plsc.load_gather(a_v, [jnp.minimum(rp[j], T - 1)])
                dl = jnp.where(lp[j] >= 0, (qq[j] + la * la) - 2.0 * (q[j] * la), inf)
                dr = jnp.where(rp[j] < l_vec, (qq[j] + ra * ra) - 2.0 * (q[j] * ra), inf)
                tl = dl <= dr
                plsc.store_scatter(
                    out_v, [qidx[j], jnp.full((LANES,), st, jnp.int32)],
                    jnp.where(tl, lp[j], rp[j]))
                lp[j] = jnp.where(tl, lp[j] - 1, lp[j])
                rp[j] = jnp.where(tl, rp[j], rp[j] + 1)
        return carry

    plsc.parallel_loop(0, CHUNKS // ILP, unroll=2, carry=jnp.int32(0))(
        lambda i, c: chunk(i, c))
    pltpu.sync_copy(out_v, out_hbm.at[vid, pl.ds(q0, QPW)])


def kernel(query_t, vid_timestamps, time_masks, k):
    q = query_t[..., 0]
    a = vid_timestamps[..., 0]
    lengths = jnp.sum(time_masks[:, :, 0], axis=1, dtype=jnp.int32)

    mesh = plsc.VectorSubcoreMesh(core_axis_name="c", subcore_axis_name="s")
    fn = pl.kernel(
        _tec_body,
        out_type=jax.ShapeDtypeStruct((V, N, K), jnp.int32),
        mesh=mesh,
        compiler_params=pltpu.CompilerParams(needs_layout_passes=False),
        scratch_types=[
            pltpu.VMEM((QPW,), jnp.float32),
            pltpu.VMEM((T,), jnp.float32),
            pltpu.VMEM((V,), jnp.int32),
            pltpu.VMEM((QPW, K), jnp.int32),
        ],
    )
    return fn(q, a, lengths)


# speedup vs baseline: 352.4625x; 1.0154x over previous
import functools

import jax
import jax.numpy as jnp
from jax import lax
from jax.experimental import pallas as pl
from jax.experimental.pallas import tpu as pltpu
from jax.experimental.pallas import tpu_sc as plsc

V, T, N, K = 16, 4096, 1024, 16
LANES = 16
NWORKERS = 32
QPW = V * N // NWORKERS
CHUNKS = QPW // LANES


def _tec_body(q_hbm, a_hbm, len_hbm, out_hbm, q_v, a_v, len_v, out_v):
    c = lax.axis_index("c")
    s = lax.axis_index("s")
    wid = s * 2 + c
    vid = wid // 2
    half = wid % 2
    q0 = half * QPW

    pltpu.sync_copy(a_hbm.at[vid], a_v)
    pltpu.sync_copy(q_hbm.at[vid, pl.ds(q0, QPW)], q_v)
    pltpu.sync_copy(len_hbm, len_v)

    lane = lax.iota(jnp.int32, LANES)
    zero = jnp.zeros((LANES,), jnp.int32)
    l_vec = plsc.load_gather(len_v, [jnp.full((LANES,), vid, jnp.int32)])
    inf = jnp.float32(jnp.inf)

    ILP = 2

    def chunk(i, carry):
        qidx = [i * (LANES * ILP) + j * LANES + lane for j in range(ILP)]
        q = [plsc.load_gather(q_v, [qidx[j]]) for j in range(ILP)]
        qq = [q[j] * q[j] for j in range(ILP)]

        pos = [jnp.zeros((LANES,), jnp.int32) for _ in range(ILP)]
        b = T // 2
        while b >= 1:
            for j in range(ILP):
                av = plsc.load_gather(a_v, [pos[j] + (b - 1)])
                pos[j] = jnp.where(av < q[j], pos[j] + b, pos[j])
            b //= 2
        pos = [jnp.minimum(p, l_vec) for p in pos]

        lp = [p - 1 for p in pos]
        rp = list(pos)
        dl = []
        dr = []
        for j in range(ILP):
            la = plsc.load_gather(a_v, [jnp.maximum(lp[j], 0)])
            ra = plsc.load_gather(a_v, [jnp.minimum(rp[j], T - 1)])
            dl.append(jnp.where(lp[j] >= 0, (qq[j] + la * la) - 2.0 * (q[j] * la), inf))
            dr.append(jnp.where(rp[j] < l_vec, (qq[j] + ra * ra) - 2.0 * (q[j] * ra), inf))
        for st in range(K):
            for j in range(ILP):
                tl = dl[j] <= dr[j]
                plsc.store_scatter(
                    out_v, [qidx[j], jnp.full((LANES,), st, jnp.int32)],
                    jnp.where(tl, lp[j], rp[j]))
                lp[j] = jnp.where(tl, lp[j] - 1, lp[j])
                rp[j] = jnp.where(tl, rp[j], rp[j] + 1)
                if st == K - 1:
                    continue
                padv = jnp.where(tl, lp[j], rp[j])
                na = plsc.load_gather(a_v, [jnp.clip(padv, 0, T - 1)])
                nd = (qq[j] + na * na) - 2.0 * (q[j] * na)
                nd = jnp.where(jnp.where(tl, lp[j] >= 0, rp[j] < l_vec), nd, inf)
                dl[j] = jnp.where(tl, nd, dl[j])
                dr[j] = jnp.where(tl, dr[j], nd)
        return carry

    plsc.parallel_loop(0, CHUNKS // ILP, unroll=2, carry=jnp.int32(0))(
        lambda i, c: chunk(i, c))
    pltpu.sync_copy(out_v, out_hbm.at[vid, pl.ds(q0, QPW)])


def kernel(query_t, vid_timestamps, time_masks, k):
    q = query_t[..., 0]
    a = vid_timestamps[..., 0]
    lengths = jnp.sum(time_masks[:, :, 0], axis=1, dtype=jnp.int32)

    mesh = plsc.VectorSubcoreMesh(core_axis_name="c", subcore_axis_name="s")
    fn = pl.kernel(
        _tec_body,
        out_type=jax.ShapeDtypeStruct((V, N, K), jnp.int32),
        mesh=mesh,
        compiler_params=pltpu.CompilerParams(needs_layout_passes=False),
        scratch_types=[
            pltpu.VMEM((QPW,), jnp.float32),
            pltpu.VMEM((T,), jnp.float32),
            pltpu.VMEM((V,), jnp.int32),
            pltpu.VMEM((QPW, K), jnp.int32),
        ],
    )
    return fn(q, a, lengths)


# speedup vs baseline: 352.7463x; 1.0008x over previous
import functools

import jax
import jax.numpy as jnp
from jax import lax
from jax.experimental import pallas as pl
from jax.experimental.pallas import tpu as pltpu
from jax.experimental.pallas import tpu_sc as plsc

V, T, N, K = 16, 4096, 1024, 16
LANES = 16
NWORKERS = 32
QPW = V * N // NWORKERS
CHUNKS = QPW // LANES


def _tec_body(q_hbm, a_hbm, len_hbm, out_hbm, q_v, a_v, len_v, out_v):
    c = lax.axis_index("c")
    s = lax.axis_index("s")
    wid = s * 2 + c
    vid = wid // 2
    half = wid % 2
    q0 = half * QPW

    pltpu.sync_copy(a_hbm.at[vid], a_v)
    pltpu.sync_copy(q_hbm.at[vid, pl.ds(q0, QPW)], q_v)
    pltpu.sync_copy(len_hbm, len_v)

    lane = lax.iota(jnp.int32, LANES)
    zero = jnp.zeros((LANES,), jnp.int32)
    l_vec = plsc.load_gather(len_v, [jnp.full((LANES,), vid, jnp.int32)])
    inf = jnp.float32(jnp.inf)

    ILP = 2

    def chunk(i, carry):
        qidx = [i * (LANES * ILP) + j * LANES + lane for j in range(ILP)]
        q = [plsc.load_gather(q_v, [qidx[j]]) for j in range(ILP)]
        qq = [q[j] * q[j] for j in range(ILP)]

        pos = [jnp.zeros((LANES,), jnp.int32) for _ in range(ILP)]
        b = T // 2
        while b >= 1:
            for j in range(ILP):
                av = plsc.load_gather(a_v, [pos[j] + (b - 1)])
                pos[j] = jnp.where(av < q[j], pos[j] + b, pos[j])
            b //= 2
        pos = [jnp.minimum(p, l_vec) for p in pos]

        lp = [p - 1 for p in pos]
        rp = list(pos)
        dl = []
        dr = []
        for j in range(ILP):
            la = plsc.load_gather(a_v, [jnp.maximum(lp[j], 0)])
            ra = plsc.load_gather(a_v, [jnp.minimum(rp[j], T - 1)])
            dl.append(jnp.where(lp[j] >= 0, (qq[j] + la * la) - 2.0 * (q[j] * la), inf))
            dr.append(jnp.where(rp[j] < l_vec, (qq[j] + ra * ra) - 2.0 * (q[j] * ra), inf))
        for st in range(K):
            for j in range(ILP):
                tl = dl[j] <= dr[j]
                plsc.store_scatter(
                    out_v, [qidx[j], jnp.full((LANES,), st, jnp.int32)],
                    jnp.where(tl, lp[j], rp[j]))
                lp[j] = jnp.where(tl, lp[j] - 1, lp[j])
                rp[j] = jnp.where(tl, rp[j], rp[j] + 1)
                if st == K - 1:
                    continue
                padv = jnp.where(tl, lp[j], rp[j])
                na = plsc.load_gather(a_v, [jnp.clip(padv, 0, T - 1)])
                nd = (qq[j] + na * na) - 2.0 * (q[j] * na)
                nd = jnp.where(jnp.where(tl, lp[j] >= 0, rp[j] < l_vec), nd, inf)
                dl[j] = jnp.where(tl, nd, dl[j])
                dr[j] = jnp.where(tl, dr[j], nd)
        return carry

    plsc.parallel_loop(0, CHUNKS // ILP, unroll=4, carry=jnp.int32(0))(
        lambda i, c: chunk(i, c))
    pltpu.sync_copy(out_v, out_hbm.at[vid, pl.ds(q0, QPW)])


def kernel(query_t, vid_timestamps, time_masks, k):
    q = query_t[..., 0]
    a = vid_timestamps[..., 0]
    lengths = jnp.sum(time_masks[:, :, 0], axis=1, dtype=jnp.int32)

    mesh = plsc.VectorSubcoreMesh(core_axis_name="c", subcore_axis_name="s")
    fn = pl.kernel(
        _tec_body,
        out_type=jax.ShapeDtypeStruct((V, N, K), jnp.int32),
        mesh=mesh,
        compiler_params=pltpu.CompilerParams(needs_layout_passes=False),
        scratch_types=[
            pltpu.VMEM((QPW,), jnp.float32),
            pltpu.VMEM((T,), jnp.float32),
            pltpu.VMEM((V,), jnp.int32),
            pltpu.VMEM((QPW, K), jnp.int32),
        ],
    )
    return fn(q, a, lengths)


# speedup vs baseline: 357.8880x; 1.0146x over previous
import functools

import jax
import jax.numpy as jnp
from jax import lax
from jax.experimental import pallas as pl
from jax.experimental.pallas import tpu as pltpu
from jax.experimental.pallas import tpu_sc as plsc

V, T, N, K = 16, 4096, 1024, 16
LANES = 16
NWORKERS = 32
QPW = V * N // NWORKERS
CHUNKS = QPW // LANES
PAD = 128
PTOT = PAD + T + 128


def _tec_body(q_hbm, a_hbm, len_hbm, out_hbm, q_v, p_v, len_v, out_v):
    c = lax.axis_index("c")
    s = lax.axis_index("s")
    wid = s * 2 + c
    vid = wid // 2
    half = wid % 2
    q0 = half * QPW

    pltpu.sync_copy(a_hbm.at[vid], p_v.at[pl.ds(PAD, T)])
    pltpu.sync_copy(q_hbm.at[vid, pl.ds(q0, QPW)], q_v)
    pltpu.sync_copy(len_hbm, len_v)

    lane = lax.iota(jnp.int32, LANES)
    l_vec = plsc.load_gather(len_v, [jnp.full((LANES,), vid, jnp.int32)])

    neg = jnp.full((LANES,), -1.0e9, jnp.float32)
    post = jnp.full((LANES,), 1.0e9, jnp.float32)
    p_v[pl.ds(PAD - LANES, LANES)] = neg
    plsc.store_scatter(p_v, [PAD + l_vec + lane], post)

    def fill(i, carry):
        idx = jnp.minimum(PAD + l_vec + i * LANES + lane, PTOT - 1)
        plsc.store_scatter(p_v, [idx], post)
        return carry

    n_fill = (PTOT - PAD - l_vec[0] + LANES - 1) // LANES
    lax.fori_loop(1, n_fill, fill, jnp.int32(0))

    ILP = 2

    def chunk(i, carry):
        qidx = [i * (LANES * ILP) + j * LANES + lane for j in range(ILP)]
        q = [plsc.load_gather(q_v, [qidx[j]]) for j in range(ILP)]
        qq = [q[j] * q[j] for j in range(ILP)]
        q2 = [q[j] + q[j] for j in range(ILP)]

        def dist(j, a):
            return (qq[j] + a * a) - q2[j] * a

        pos = [jnp.full((LANES,), PAD, jnp.int32) for _ in range(ILP)]
        b = T // 2
        while b >= 1:
            for j in range(ILP):
                av = plsc.load_gather(p_v, [pos[j] + (b - 1)])
                pos[j] = jnp.where(av < q[j], pos[j] + b, pos[j])
            b //= 2

        lp = [p - 1 for p in pos]
        rp = list(pos)
        dl = [dist(j, plsc.load_gather(p_v, [lp[j]])) for j in range(ILP)]
        dr = [dist(j, plsc.load_gather(p_v, [rp[j]])) for j in range(ILP)]
        for st in range(K):
            for j in range(ILP):
                tl = dl[j] <= dr[j]
                sel = jnp.where(tl, lp[j], rp[j])
                plsc.store_scatter(
                    out_v, [qidx[j], jnp.full((LANES,), st, jnp.int32)],
                    sel - PAD)
                if st == K - 1:
                    continue
                adv = jnp.where(tl, sel - 1, sel + 1)
                lp[j] = jnp.where(tl, adv, lp[j])
                rp[j] = jnp.where(tl, rp[j], adv)
                nd = dist(j, plsc.load_gather(p_v, [adv]))
                dl[j] = jnp.where(tl, nd, dl[j])
                dr[j] = jnp.where(tl, dr[j], nd)
        return carry

    plsc.parallel_loop(0, CHUNKS // ILP, unroll=4, carry=jnp.int32(0))(
        lambda i, c: chunk(i, c))
    pltpu.sync_copy(out_v, out_hbm.at[vid, pl.ds(q0, QPW)])


def kernel(query_t, vid_timestamps, time_masks, k):
    q = query_t[..., 0]
    a = vid_timestamps[..., 0]
    lengths = jnp.sum(time_masks[:, :, 0], axis=1, dtype=jnp.int32)

    mesh = plsc.VectorSubcoreMesh(core_axis_name="c", subcore_axis_name="s")
    fn = pl.kernel(
        _tec_body,
        out_type=jax.ShapeDtypeStruct((V, N, K), jnp.int32),
        mesh=mesh,
        compiler_params=pltpu.CompilerParams(needs_layout_passes=False),
        scratch_types=[
            pltpu.VMEM((QPW,), jnp.float32),
            pltpu.VMEM((PTOT,), jnp.float32),
            pltpu.VMEM((V,), jnp.int32),
            pltpu.VMEM((QPW, K), jnp.int32),
        ],
    )
    return fn(q, a, lengths)
